# baseline (speedup 1.0000x reference)
import jax
import jax.numpy as jnp
import numpy as np
from jax import lax
from jax.experimental import pallas as pl
from jax.experimental.pallas import tpu as pltpu

EPS = 1e-08
_LOG_EPS = float(np.log(np.float32(EPS)))


def _write_head_kernel(mem_ref, ctrl_ref, out_ref):
    memt = mem_ref[0]
    ctrl = ctrl_ref[0]
    w, c = memt.shape

    keys = ctrl[:, 0:w]
    erase = jax.nn.sigmoid(ctrl[:, w:2 * w])
    add = ctrl[:, 2 * w:3 * w]
    beta = jax.nn.softplus(ctrl[:, -3:-2])
    alloc_gate = jax.nn.sigmoid(ctrl[:, -2:-1])
    write_gate = jax.nn.sigmoid(ctrl[:, -1:])

    dot = jnp.dot(keys, memt)
    ones_row = jnp.ones((1, w), dtype=memt.dtype)
    nrm2 = jnp.dot(ones_row, memt * memt)

    key_norm = jnp.sqrt(jnp.sum(keys * keys))
    scores = (dot * beta) / (key_norm * jnp.sqrt(nrm2) + EPS)

    smax = jnp.max(scores)
    e = jnp.exp(scores - smax)
    content_w = e * ((write_gate * (1.0 - alloc_gate)) / jnp.sum(e))

    idx = lax.broadcasted_iota(jnp.int32, (1, c), 1).astype(jnp.float32)
    alloc = (1.0 - EPS) * jnp.exp(idx * _LOG_EPS)
    ww = content_w + (write_gate * alloc_gate) * alloc

    ecol = jnp.broadcast_to(erase, (w, w)).T[:, 0:1]
    acol = jnp.broadcast_to(add, (w, w)).T[:, 0:1]
    out_ref[0] = memt - ww * (memt * ecol - acol)


def kernel(memory, controls, read_weights):
    b, c, w = memory.shape
    n = controls.shape[-1]
    memt = jnp.swapaxes(memory, 1, 2)
    ctrl3 = controls.reshape(b, 1, n)
    out_t = pl.pallas_call(
        _write_head_kernel,
        grid=(b,),
        in_specs=[
            pl.BlockSpec((1, w, c), lambda i: (i, 0, 0)),
            pl.BlockSpec((1, 1, n), lambda i: (i, 0, 0)),
        ],
        out_specs=pl.BlockSpec((1, w, c), lambda i: (i, 0, 0)),
        out_shape=jax.ShapeDtypeStruct((b, w, c), memory.dtype),
        compiler_params=pltpu.CompilerParams(
            dimension_semantics=("parallel",),
        ),
    )(memt, ctrl3)
    return jnp.swapaxes(out_t, 1, 2)


# speedup vs baseline: 1.1484x; 1.1484x over previous
import jax
import jax.numpy as jnp
import numpy as np
from jax import lax
from jax.experimental import pallas as pl
from jax.experimental.pallas import tpu as pltpu

EPS = 1e-08
_LOG_EPS = float(np.log(np.float32(EPS)))
_BB = 2


def _write_head_kernel(mem_ref, ctrl_ref, out_ref):
    memt = mem_ref[...]
    ctrl = ctrl_ref[...]
    bb, w, c = memt.shape

    keys = ctrl[:, :, 0:w]
    erase = jax.nn.sigmoid(ctrl[:, :, w:2 * w])
    add = ctrl[:, :, 2 * w:3 * w]
    beta = jax.nn.softplus(ctrl[:, :, -3:-2])
    alloc_gate = jax.nn.sigmoid(ctrl[:, :, -2:-1])
    write_gate = jax.nn.sigmoid(ctrl[:, :, -1:])

    dims = (((2,), (1,)), ((0,), (0,)))
    dot = lax.dot_general(keys, memt, dims)
    ones_row = jnp.ones((bb, 1, w), dtype=memt.dtype)
    nrm2 = lax.dot_general(ones_row, memt * memt, dims)

    key_norm = jnp.sqrt(jnp.sum(keys * keys, axis=-1, keepdims=True))
    scores = (dot * beta) / (key_norm * jnp.sqrt(nrm2) + EPS)

    smax = jnp.max(scores, axis=(1, 2), keepdims=True)
    e = jnp.exp(scores - smax)
    ssum = jnp.sum(e, axis=(1, 2), keepdims=True)
    content_w = e * ((write_gate * (1.0 - alloc_gate)) / ssum)

    idx = lax.broadcasted_iota(jnp.int32, (1, 1, c), 2).astype(jnp.float32)
    alloc = (1.0 - EPS) * jnp.exp(idx * _LOG_EPS)
    ww = content_w + (write_gate * alloc_gate) * alloc

    ecol = jnp.swapaxes(jnp.broadcast_to(erase, (bb, w, w)), 1, 2)[:, :, 0:1]
    acol = jnp.swapaxes(jnp.broadcast_to(add, (bb, w, w)), 1, 2)[:, :, 0:1]
    out_ref[...] = memt - ww * (memt * ecol - acol)


def kernel(memory, controls, read_weights):
    b, c, w = memory.shape
    n = controls.shape[-1]
    memt = jnp.swapaxes(memory, 1, 2)
    ctrl3 = controls.reshape(b, 1, n)
    out_t = pl.pallas_call(
        _write_head_kernel,
        grid=(b // _BB,),
        in_specs=[
            pl.BlockSpec((_BB, w, c), lambda i: (i, 0, 0)),
            pl.BlockSpec((_BB, 1, n), lambda i: (i, 0, 0)),
        ],
        out_specs=pl.BlockSpec((_BB, w, c), lambda i: (i, 0, 0)),
        out_shape=jax.ShapeDtypeStruct((b, w, c), memory.dtype),
        compiler_params=pltpu.CompilerParams(
            dimension_semantics=("parallel",),
        ),
    )(memt, ctrl3)
    return jnp.swapaxes(out_t, 1, 2)
